# baseline (speedup 1.0000x reference)
import functools

import jax
import jax.numpy as jnp
from jax import lax
from jax.experimental import pallas as pl
from jax.experimental.pallas import tpu as pltpu
from jax.experimental.pallas import tpu_sc as plsc

BATCH = 1024
LATENT = 256
K_TOTAL = 100000
BK = 1408
NFULL = K_TOTAL // BK
TAIL_W = K_TOTAL - NFULL * BK
TAIL_A = TAIL_W + 1


NSPLIT = 2
ROWS = BATCH // NSPLIT
WROWS = LATENT // NSPLIT


def _mm_body(x_hbm, w_hbm, b_ref, o_ref,
             a_bufs, w_bufs, a_tail, w_tail, carry_ref,
             a_sems, w_sems, t_sems):
    k = pl.program_id(0)
    slot = jax.lax.rem(k, 2)
    nxt = jax.lax.rem(k + 1, 2)

    def start_full(i, s):
        for q in range(NSPLIT):
            pltpu.make_async_copy(
                x_hbm.at[pl.ds(q * ROWS, ROWS), pl.ds(i * BK, BK)],
                a_bufs.at[s, pl.ds(q * ROWS, ROWS)], a_sems.at[s, q],
            ).start(priority=q)
            pltpu.make_async_copy(
                w_hbm.at[pl.ds(q * WROWS, WROWS), pl.ds(i * BK, BK)],
                w_bufs.at[s, pl.ds(q * WROWS, WROWS)], w_sems.at[s, q],
            ).start(priority=q)

    def wait_full(s):
        for q in range(NSPLIT):
            pltpu.make_async_copy(
                x_hbm.at[pl.ds(0, ROWS), pl.ds(0, BK)],
                a_bufs.at[s, pl.ds(0, ROWS)], a_sems.at[s, q],
            ).wait()
            pltpu.make_async_copy(
                w_hbm.at[pl.ds(0, WROWS), pl.ds(0, BK)],
                w_bufs.at[s, pl.ds(0, WROWS)], w_sems.at[s, q],
            ).wait()

    @pl.when(k == 0)
    def _():
        carry_ref[...] = jnp.zeros((LATENT, 1), jnp.float32)
        start_full(0, 0)

    @pl.when(k + 1 < NFULL)
    def _():
        start_full(k + 1, nxt)

    @pl.when(k + 1 == NFULL)
    def _():
        pltpu.make_async_copy(
            x_hbm.at[:, pl.ds(NFULL * BK, TAIL_A)], a_tail, t_sems.at[0]
        ).start(priority=0)
        pltpu.make_async_copy(
            w_hbm.at[:, pl.ds(NFULL * BK, TAIL_W)], w_tail, t_sems.at[1]
        ).start(priority=1)

    carry_col = carry_ref[...]

    @pl.when(k < NFULL)
    def _():
        wait_full(slot)
        wk = w_bufs[slot]
        rolled = pltpu.roll(wk, 1, 1)
        lane = lax.broadcasted_iota(jnp.int32, (LATENT, BK), 1)
        wshift = jnp.where(lane == 0, carry_col, rolled)
        carry_ref[...] = wk[:, BK - 1:BK]
        acc = lax.dot_general(
            a_bufs[slot].astype(jnp.bfloat16), wshift.astype(jnp.bfloat16),
            (((1,), (1,)), ((), ())),
            preferred_element_type=jnp.float32,
        )

        @pl.when(k == 0)
        def _():
            o_ref[...] = acc + b_ref[...]

        @pl.when(k > 0)
        def _():
            o_ref[...] += acc

    @pl.when(k == NFULL)
    def _():
        pltpu.make_async_copy(
            x_hbm.at[:, pl.ds(NFULL * BK, TAIL_A)], a_tail, t_sems.at[0]
        ).wait()
        pltpu.make_async_copy(
            w_hbm.at[:, pl.ds(NFULL * BK, TAIL_W)], w_tail, t_sems.at[1]
        ).wait()
        wsh = jnp.concatenate(
            [carry_col, w_tail[...]], axis=1)
        o_ref[...] += lax.dot_general(
            a_tail[...].astype(jnp.bfloat16), wsh.astype(jnp.bfloat16),
            (((1,), (1,)), ((), ())),
            preferred_element_type=jnp.float32,
        )


def _matmul(input_data, W, b2d):
    return pl.pallas_call(
        _mm_body,
        grid=(NFULL + 1,),
        in_specs=[
            pl.BlockSpec(memory_space=pltpu.MemorySpace.HBM),
            pl.BlockSpec(memory_space=pltpu.MemorySpace.HBM),
            pl.BlockSpec((1, LATENT), lambda k: (0, 0)),
        ],
        out_specs=pl.BlockSpec((BATCH, LATENT), lambda k: (0, 0)),
        out_shape=jax.ShapeDtypeStruct((BATCH, LATENT), jnp.float32),
        scratch_shapes=[
            pltpu.VMEM((2, BATCH, BK), jnp.float32),
            pltpu.VMEM((2, LATENT, BK), jnp.float32),
            pltpu.VMEM((BATCH, TAIL_A), jnp.float32),
            pltpu.VMEM((LATENT, TAIL_W), jnp.float32),
            pltpu.VMEM((LATENT, 1), jnp.float32),
            pltpu.SemaphoreType.DMA((2, NSPLIT)),
            pltpu.SemaphoreType.DMA((2, NSPLIT)),
            pltpu.SemaphoreType.DMA((2,)),
        ],
        compiler_params=pltpu.CompilerParams(
            dimension_semantics=("arbitrary",),
        ),
    )(input_data, W, b2d)


def _make_sc_gather():
    info = plsc.get_sparse_core_info()
    nc, ns = info.num_cores, info.num_subcores
    nw = nc * ns
    b_per_w = BATCH // nw
    mesh = plsc.VectorSubcoreMesh(core_axis_name="c", subcore_axis_name="s")

    @functools.partial(
        pl.kernel,
        mesh=mesh,
        out_type=jax.ShapeDtypeStruct((BATCH, LATENT), jnp.float32),
        scratch_types=[
            pltpu.VMEM((b_per_w,), jnp.int32),
            pltpu.VMEM((b_per_w, LATENT), jnp.float32),
            pltpu.SemaphoreType.DMA,
        ],
    )
    def gather(table_hbm, idx_hbm, out_hbm, idx_v, rows_v, sem):
        wid = lax.axis_index("s") * nc + lax.axis_index("c")
        base = wid * b_per_w
        pltpu.sync_copy(idx_hbm.at[pl.ds(base, b_per_w)], idx_v)
        pltpu.async_copy(table_hbm.at[idx_v], rows_v, sem).wait()
        pltpu.sync_copy(rows_v, out_hbm.at[pl.ds(base, b_per_w)])

    return gather


_sc_gather = None


def kernel(input_data, users_embedding, W, b):
    global _sc_gather
    if _sc_gather is None:
        _sc_gather = _make_sc_gather()
    user_ids = input_data[:, 0].astype(jnp.int32)
    users_embed = _sc_gather(users_embedding, user_ids)
    mm = _matmul(input_data, W, b.reshape(1, LATENT))
    return mm + users_embed


# speedup vs baseline: 1.0756x; 1.0756x over previous
import functools

import jax
import jax.numpy as jnp
from jax import lax
from jax.experimental import pallas as pl
from jax.experimental.pallas import tpu as pltpu
from jax.experimental.pallas import tpu_sc as plsc

BATCH = 1024
LATENT = 256
K_TOTAL = 100000
BK = 1408
NFULL = K_TOTAL // BK
TAIL_W = K_TOTAL - NFULL * BK
TAIL_A = TAIL_W + 1

NBUF = 3
NSA = 4
NSW = 2
AROWS = BATCH // NSA
WROWS = LATENT // NSW


def _mm_body(x_hbm, w_hbm, b_ref, o_ref,
             a_bufs, w_bufs, a_tail, w_tail, carry_ref,
             a_sems, w_sems, t_sems):
    k = pl.program_id(0)
    slot = jax.lax.rem(k, NBUF)

    def start_full(i, s):
        for q in range(NSA):
            pltpu.make_async_copy(
                x_hbm.at[pl.ds(q * AROWS, AROWS), pl.ds(i * BK, BK)],
                a_bufs.at[s, pl.ds(q * AROWS, AROWS)], a_sems.at[s, q],
            ).start(priority=q % 2)
        for q in range(NSW):
            pltpu.make_async_copy(
                w_hbm.at[pl.ds(q * WROWS, WROWS), pl.ds(i * BK, BK)],
                w_bufs.at[s, pl.ds(q * WROWS, WROWS)], w_sems.at[s, q],
            ).start(priority=q % 2)

    def wait_full(s):
        for q in range(NSA):
            pltpu.make_async_copy(
                x_hbm.at[pl.ds(0, AROWS), pl.ds(0, BK)],
                a_bufs.at[s, pl.ds(0, AROWS)], a_sems.at[s, q],
            ).wait()
        for q in range(NSW):
            pltpu.make_async_copy(
                w_hbm.at[pl.ds(0, WROWS), pl.ds(0, BK)],
                w_bufs.at[s, pl.ds(0, WROWS)], w_sems.at[s, q],
            ).wait()

    @pl.when(k == 0)
    def _():
        carry_ref[...] = jnp.zeros((LATENT, 1), jnp.float32)
        for i in range(NBUF - 1):
            start_full(i, i)

    @pl.when(k + NBUF - 1 < NFULL)
    def _():
        start_full(k + NBUF - 1, jax.lax.rem(k + NBUF - 1, NBUF))

    @pl.when(k + NBUF - 1 == NFULL)
    def _():
        pltpu.make_async_copy(
            x_hbm.at[:, pl.ds(NFULL * BK, TAIL_A)], a_tail, t_sems.at[0]
        ).start(priority=0)
        pltpu.make_async_copy(
            w_hbm.at[:, pl.ds(NFULL * BK, TAIL_W)], w_tail, t_sems.at[1]
        ).start(priority=1)

    carry_col = carry_ref[...]

    @pl.when(k < NFULL)
    def _():
        wait_full(slot)
        wk = w_bufs[slot]
        rolled = pltpu.roll(wk, 1, 1)
        lane = lax.broadcasted_iota(jnp.int32, (LATENT, BK), 1)
        wshift = jnp.where(lane == 0, carry_col, rolled)
        carry_ref[...] = wk[:, BK - 1:BK]
        acc = lax.dot_general(
            a_bufs[slot].astype(jnp.bfloat16), wshift.astype(jnp.bfloat16),
            (((1,), (1,)), ((), ())),
            preferred_element_type=jnp.float32,
        )

        @pl.when(k == 0)
        def _():
            o_ref[...] = acc + b_ref[...]

        @pl.when(k > 0)
        def _():
            o_ref[...] += acc

    @pl.when(k == NFULL)
    def _():
        pltpu.make_async_copy(
            x_hbm.at[:, pl.ds(NFULL * BK, TAIL_A)], a_tail, t_sems.at[0]
        ).wait()
        pltpu.make_async_copy(
            w_hbm.at[:, pl.ds(NFULL * BK, TAIL_W)], w_tail, t_sems.at[1]
        ).wait()
        wsh = jnp.concatenate(
            [carry_col, w_tail[...]], axis=1)
        o_ref[...] += lax.dot_general(
            a_tail[...].astype(jnp.bfloat16), wsh.astype(jnp.bfloat16),
            (((1,), (1,)), ((), ())),
            preferred_element_type=jnp.float32,
        )


def _matmul(input_data, W, b2d):
    return pl.pallas_call(
        _mm_body,
        grid=(NFULL + 1,),
        in_specs=[
            pl.BlockSpec(memory_space=pltpu.MemorySpace.HBM),
            pl.BlockSpec(memory_space=pltpu.MemorySpace.HBM),
            pl.BlockSpec((1, LATENT), lambda k: (0, 0)),
        ],
        out_specs=pl.BlockSpec((BATCH, LATENT), lambda k: (0, 0)),
        out_shape=jax.ShapeDtypeStruct((BATCH, LATENT), jnp.float32),
        scratch_shapes=[
            pltpu.VMEM((NBUF, BATCH, BK), jnp.float32),
            pltpu.VMEM((NBUF, LATENT, BK), jnp.float32),
            pltpu.VMEM((BATCH, TAIL_A), jnp.float32),
            pltpu.VMEM((LATENT, TAIL_W), jnp.float32),
            pltpu.VMEM((LATENT, 1), jnp.float32),
            pltpu.SemaphoreType.DMA((NBUF, NSA)),
            pltpu.SemaphoreType.DMA((NBUF, NSW)),
            pltpu.SemaphoreType.DMA((2,)),
        ],
        compiler_params=pltpu.CompilerParams(
            dimension_semantics=("arbitrary",),
        ),
    )(input_data, W, b2d)


def _make_sc_gather():
    info = plsc.get_sparse_core_info()
    nc, ns = info.num_cores, info.num_subcores
    nw = nc * ns
    b_per_w = BATCH // nw
    mesh = plsc.VectorSubcoreMesh(core_axis_name="c", subcore_axis_name="s")

    @functools.partial(
        pl.kernel,
        mesh=mesh,
        out_type=jax.ShapeDtypeStruct((BATCH, LATENT), jnp.float32),
        scratch_types=[
            pltpu.VMEM((b_per_w,), jnp.int32),
            pltpu.VMEM((b_per_w, LATENT), jnp.float32),
            pltpu.SemaphoreType.DMA,
        ],
    )
    def gather(table_hbm, idx_hbm, out_hbm, idx_v, rows_v, sem):
        wid = lax.axis_index("s") * nc + lax.axis_index("c")
        base = wid * b_per_w
        pltpu.sync_copy(idx_hbm.at[pl.ds(base, b_per_w)], idx_v)
        pltpu.async_copy(table_hbm.at[idx_v], rows_v, sem).wait()
        pltpu.sync_copy(rows_v, out_hbm.at[pl.ds(base, b_per_w)])

    return gather


_sc_gather = None


def kernel(input_data, users_embedding, W, b):
    global _sc_gather
    if _sc_gather is None:
        _sc_gather = _make_sc_gather()
    user_ids = input_data[:, 0].astype(jnp.int32)
    users_embed = _sc_gather(users_embedding, user_ids)
    mm = _matmul(input_data, W, b.reshape(1, LATENT))
    return mm + users_embed


# speedup vs baseline: 3.0790x; 2.8626x over previous
import functools

import jax
import jax.numpy as jnp
from jax import lax
from jax.experimental import pallas as pl
from jax.experimental.pallas import tpu as pltpu
from jax.experimental.pallas import tpu_sc as plsc

BATCH = 1024
LATENT = 256
K_TOTAL = 100000
BK = 1408
NFULL = K_TOTAL // BK
TAIL_W = K_TOTAL - NFULL * BK
TAIL_A = TAIL_W + 1

NBUF = 3
NSA = 4
NSW = 2
ABAND = BK // NSA
WBAND = BK // NSW


def _mm_body(xt_hbm, wt_hbm, b_ref, o_ref,
             a_bufs, w_bufs, a_tail, w_tail, carry_ref,
             a_sems, w_sems, t_sems):
    k = pl.program_id(0)
    slot = jax.lax.rem(k, NBUF)

    def start_full(i, s):
        for q in range(NSA):
            pltpu.make_async_copy(
                xt_hbm.at[pl.ds(i * BK + q * ABAND, ABAND), :],
                a_bufs.at[s, pl.ds(q * ABAND, ABAND)], a_sems.at[s, q],
            ).start(priority=q % 2)
        for q in range(NSW):
            pltpu.make_async_copy(
                wt_hbm.at[pl.ds(i * BK + q * WBAND, WBAND), :],
                w_bufs.at[s, pl.ds(q * WBAND, WBAND)], w_sems.at[s, q],
            ).start(priority=q % 2)

    def wait_full(s):
        for q in range(NSA):
            pltpu.make_async_copy(
                xt_hbm.at[pl.ds(0, ABAND), :],
                a_bufs.at[s, pl.ds(0, ABAND)], a_sems.at[s, q],
            ).wait()
        for q in range(NSW):
            pltpu.make_async_copy(
                wt_hbm.at[pl.ds(0, WBAND), :],
                w_bufs.at[s, pl.ds(0, WBAND)], w_sems.at[s, q],
            ).wait()

    @pl.when(k == 0)
    def _():
        carry_ref[...] = jnp.zeros((1, LATENT), jnp.float32)
        for i in range(NBUF - 1):
            start_full(i, i)

    @pl.when(k + NBUF - 1 < NFULL)
    def _():
        start_full(k + NBUF - 1, jax.lax.rem(k + NBUF - 1, NBUF))

    @pl.when(k + NBUF - 1 == NFULL)
    def _():
        pltpu.make_async_copy(
            xt_hbm.at[pl.ds(NFULL * BK, TAIL_A), :], a_tail, t_sems.at[0]
        ).start(priority=0)
        pltpu.make_async_copy(
            wt_hbm.at[pl.ds(NFULL * BK, TAIL_W), :], w_tail, t_sems.at[1]
        ).start(priority=1)

    carry_row = carry_ref[...]

    @pl.when(k < NFULL)
    def _():
        wait_full(slot)
        wk = w_bufs[slot]
        rolled = pltpu.roll(wk, 1, 0)
        sub = lax.broadcasted_iota(jnp.int32, (BK, LATENT), 0)
        wshift = jnp.where(sub == 0, carry_row, rolled)
        carry_ref[...] = wk[BK - 1:BK, :]
        acc = lax.dot_general(
            a_bufs[slot].astype(jnp.bfloat16), wshift.astype(jnp.bfloat16),
            (((0,), (0,)), ((), ())),
            preferred_element_type=jnp.float32,
        )

        @pl.when(k == 0)
        def _():
            o_ref[...] = acc + b_ref[...]

        @pl.when(k > 0)
        def _():
            o_ref[...] += acc

    @pl.when(k == NFULL)
    def _():
        pltpu.make_async_copy(
            xt_hbm.at[pl.ds(NFULL * BK, TAIL_A), :], a_tail, t_sems.at[0]
        ).wait()
        pltpu.make_async_copy(
            wt_hbm.at[pl.ds(NFULL * BK, TAIL_W), :], w_tail, t_sems.at[1]
        ).wait()
        wsh = jnp.concatenate(
            [carry_row, w_tail[...]], axis=0)
        o_ref[...] += lax.dot_general(
            a_tail[...].astype(jnp.bfloat16), wsh.astype(jnp.bfloat16),
            (((0,), (0,)), ((), ())),
            preferred_element_type=jnp.float32,
        )


def _matmul(xt, wt, b2d):
    return pl.pallas_call(
        _mm_body,
        grid=(NFULL + 1,),
        in_specs=[
            pl.BlockSpec(memory_space=pltpu.MemorySpace.HBM),
            pl.BlockSpec(memory_space=pltpu.MemorySpace.HBM),
            pl.BlockSpec((1, LATENT), lambda k: (0, 0)),
        ],
        out_specs=pl.BlockSpec((BATCH, LATENT), lambda k: (0, 0)),
        out_shape=jax.ShapeDtypeStruct((BATCH, LATENT), jnp.float32),
        scratch_shapes=[
            pltpu.VMEM((NBUF, BK, BATCH), jnp.float32),
            pltpu.VMEM((NBUF, BK, LATENT), jnp.float32),
            pltpu.VMEM((TAIL_A, BATCH), jnp.float32),
            pltpu.VMEM((TAIL_W, LATENT), jnp.float32),
            pltpu.VMEM((1, LATENT), jnp.float32),
            pltpu.SemaphoreType.DMA((NBUF, NSA)),
            pltpu.SemaphoreType.DMA((NBUF, NSW)),
            pltpu.SemaphoreType.DMA((2,)),
        ],
        compiler_params=pltpu.CompilerParams(
            dimension_semantics=("arbitrary",),
        ),
    )(xt, wt, b2d)


def _make_sc_gather():
    info = plsc.get_sparse_core_info()
    nc, ns = info.num_cores, info.num_subcores
    nw = nc * ns
    b_per_w = BATCH // nw
    mesh = plsc.VectorSubcoreMesh(core_axis_name="c", subcore_axis_name="s")

    @functools.partial(
        pl.kernel,
        mesh=mesh,
        out_type=jax.ShapeDtypeStruct((BATCH, LATENT), jnp.float32),
        scratch_types=[
            pltpu.VMEM((b_per_w,), jnp.int32),
            pltpu.VMEM((b_per_w, LATENT), jnp.float32),
            pltpu.SemaphoreType.DMA,
        ],
    )
    def gather(table_hbm, idx_hbm, out_hbm, idx_v, rows_v, sem):
        wid = lax.axis_index("s") * nc + lax.axis_index("c")
        base = wid * b_per_w
        pltpu.sync_copy(idx_hbm.at[pl.ds(base, b_per_w)], idx_v)
        pltpu.async_copy(table_hbm.at[idx_v], rows_v, sem).wait()
        pltpu.sync_copy(rows_v, out_hbm.at[pl.ds(base, b_per_w)])

    return gather


_sc_gather = None


def kernel(input_data, users_embedding, W, b):
    global _sc_gather
    if _sc_gather is None:
        _sc_gather = _make_sc_gather()
    user_ids = input_data[:, 0].astype(jnp.int32)
    users_embed = _sc_gather(users_embedding, user_ids)
    mm = _matmul(input_data.T, W.T, b.reshape(1, LATENT))
    return mm + users_embed


# speedup vs baseline: 3.1674x; 1.0287x over previous
import functools

import jax
import jax.numpy as jnp
from jax import lax
from jax.experimental import pallas as pl
from jax.experimental.pallas import tpu as pltpu
from jax.experimental.pallas import tpu_sc as plsc

BATCH = 1024
LATENT = 256
K_TOTAL = 100000
BK = 1408
NFULL = K_TOTAL // BK
TAIL_W = K_TOTAL - NFULL * BK
TAIL_A = TAIL_W + 1

NBUF = 4
NSA = 4
NSW = 2
ABAND = BK // NSA
WBAND = BK // NSW


def _mm_body(xt_hbm, wt_hbm, b_ref, o_ref,
             a_bufs, w_bufs, a_tail, w_tail, carry_ref,
             a_sems, w_sems, t_sems):
    k = pl.program_id(0)
    slot = jax.lax.rem(k, NBUF)

    def start_full(i, s):
        for q in range(NSA):
            pltpu.make_async_copy(
                xt_hbm.at[pl.ds(i * BK + q * ABAND, ABAND), :],
                a_bufs.at[s, pl.ds(q * ABAND, ABAND)], a_sems.at[s, q],
            ).start(priority=q % 2)
        for q in range(NSW):
            pltpu.make_async_copy(
                wt_hbm.at[pl.ds(i * BK + q * WBAND, WBAND), :],
                w_bufs.at[s, pl.ds(q * WBAND, WBAND)], w_sems.at[s, q],
            ).start(priority=q % 2)

    def wait_full(s):
        for q in range(NSA):
            pltpu.make_async_copy(
                xt_hbm.at[pl.ds(0, ABAND), :],
                a_bufs.at[s, pl.ds(0, ABAND)], a_sems.at[s, q],
            ).wait()
        for q in range(NSW):
            pltpu.make_async_copy(
                wt_hbm.at[pl.ds(0, WBAND), :],
                w_bufs.at[s, pl.ds(0, WBAND)], w_sems.at[s, q],
            ).wait()

    @pl.when(k == 0)
    def _():
        carry_ref[...] = jnp.zeros((1, LATENT), jnp.float32)
        for i in range(NBUF - 1):
            start_full(i, i)

    @pl.when(k + NBUF - 1 < NFULL)
    def _():
        start_full(k + NBUF - 1, jax.lax.rem(k + NBUF - 1, NBUF))

    @pl.when(k + NBUF - 1 == NFULL)
    def _():
        pltpu.make_async_copy(
            xt_hbm.at[pl.ds(NFULL * BK, TAIL_A), :], a_tail, t_sems.at[0]
        ).start(priority=0)
        pltpu.make_async_copy(
            wt_hbm.at[pl.ds(NFULL * BK, TAIL_W), :], w_tail, t_sems.at[1]
        ).start(priority=1)

    carry_row = carry_ref[...]

    @pl.when(k < NFULL)
    def _():
        wait_full(slot)
        wk = w_bufs[slot]
        rolled = pltpu.roll(wk, 1, 0)
        sub = lax.broadcasted_iota(jnp.int32, (BK, LATENT), 0)
        wshift = jnp.where(sub == 0, carry_row, rolled)
        carry_ref[...] = wk[BK - 1:BK, :]
        acc = lax.dot_general(
            a_bufs[slot].astype(jnp.bfloat16), wshift.astype(jnp.bfloat16),
            (((0,), (0,)), ((), ())),
            preferred_element_type=jnp.float32,
        )

        @pl.when(k == 0)
        def _():
            o_ref[...] = acc + b_ref[...]

        @pl.when(k > 0)
        def _():
            o_ref[...] += acc

    @pl.when(k == NFULL)
    def _():
        pltpu.make_async_copy(
            xt_hbm.at[pl.ds(NFULL * BK, TAIL_A), :], a_tail, t_sems.at[0]
        ).wait()
        pltpu.make_async_copy(
            wt_hbm.at[pl.ds(NFULL * BK, TAIL_W), :], w_tail, t_sems.at[1]
        ).wait()
        wsh = jnp.concatenate(
            [carry_row, w_tail[...]], axis=0)
        o_ref[...] += lax.dot_general(
            a_tail[...].astype(jnp.bfloat16), wsh.astype(jnp.bfloat16),
            (((0,), (0,)), ((), ())),
            preferred_element_type=jnp.float32,
        )


def _matmul(xt, wt, b2d):
    return pl.pallas_call(
        _mm_body,
        grid=(NFULL + 1,),
        in_specs=[
            pl.BlockSpec(memory_space=pltpu.MemorySpace.HBM),
            pl.BlockSpec(memory_space=pltpu.MemorySpace.HBM),
            pl.BlockSpec((1, LATENT), lambda k: (0, 0)),
        ],
        out_specs=pl.BlockSpec((BATCH, LATENT), lambda k: (0, 0)),
        out_shape=jax.ShapeDtypeStruct((BATCH, LATENT), jnp.float32),
        scratch_shapes=[
            pltpu.VMEM((NBUF, BK, BATCH), jnp.float32),
            pltpu.VMEM((NBUF, BK, LATENT), jnp.float32),
            pltpu.VMEM((TAIL_A, BATCH), jnp.float32),
            pltpu.VMEM((TAIL_W, LATENT), jnp.float32),
            pltpu.VMEM((1, LATENT), jnp.float32),
            pltpu.SemaphoreType.DMA((NBUF, NSA)),
            pltpu.SemaphoreType.DMA((NBUF, NSW)),
            pltpu.SemaphoreType.DMA((2,)),
        ],
        compiler_params=pltpu.CompilerParams(
            dimension_semantics=("arbitrary",),
        ),
    )(xt, wt, b2d)


def _make_sc_gather():
    info = plsc.get_sparse_core_info()
    nc, ns = info.num_cores, info.num_subcores
    nw = nc * ns
    b_per_w = BATCH // nw
    mesh = plsc.VectorSubcoreMesh(core_axis_name="c", subcore_axis_name="s")

    @functools.partial(
        pl.kernel,
        mesh=mesh,
        out_type=jax.ShapeDtypeStruct((BATCH, LATENT), jnp.float32),
        scratch_types=[
            pltpu.VMEM((b_per_w,), jnp.int32),
            pltpu.VMEM((b_per_w, LATENT), jnp.float32),
            pltpu.SemaphoreType.DMA,
        ],
    )
    def gather(table_hbm, idx_hbm, out_hbm, idx_v, rows_v, sem):
        wid = lax.axis_index("s") * nc + lax.axis_index("c")
        base = wid * b_per_w
        pltpu.sync_copy(idx_hbm.at[pl.ds(base, b_per_w)], idx_v)
        pltpu.async_copy(table_hbm.at[idx_v], rows_v, sem).wait()
        pltpu.sync_copy(rows_v, out_hbm.at[pl.ds(base, b_per_w)])

    return gather


_sc_gather = None


def kernel(input_data, users_embedding, W, b):
    global _sc_gather
    if _sc_gather is None:
        _sc_gather = _make_sc_gather()
    user_ids = input_data[:, 0].astype(jnp.int32)
    users_embed = _sc_gather(users_embedding, user_ids)
    mm = _matmul(input_data.T, W.T, b.reshape(1, LATENT))
    return mm + users_embed
